# Initial kernel scaffold; baseline (speedup 1.0000x reference)
import jax
import jax.numpy as jnp
from jax.experimental import pallas as pl


def kernel(x, edge_index, W1, b1, W2, b2, W3, b3):
    raise NotImplementedError("write your pallas kernel here")


# speedup vs baseline: 24.7767x; 24.7767x over previous
import functools

import jax
import jax.numpy as jnp
from jax import lax
from jax.experimental import pallas as pl
from jax.experimental.pallas import tpu as pltpu
from jax.experimental.pallas import tpu_sc as plsc

N = 10000
E = 320000
DIN = 128
DH = 16
DOUT = 2
NC = 2
NS = 16
NW = NC * NS
EPW = E // NW
CH = 80
NCHUNK = EPW // CH
RPT = N // NS

_mesh = plsc.VectorSubcoreMesh(core_axis_name="c", subcore_axis_name="s")
_sc_params = pltpu.CompilerParams(use_tc_tiling_on_sc=False)


def _sc_degree(dst, ones, zeros):
  @functools.partial(
      pl.kernel,
      mesh=_mesh,
      out_type=jax.ShapeDtypeStruct((NC, N, DH), jnp.float32),
      compiler_params=_sc_params,
      scratch_types=[
          pltpu.VMEM((NCHUNK, CH), jnp.int32),
          pltpu.VMEM((CH, DH), jnp.float32),
          pltpu.VMEM_SHARED((N, DH), jnp.float32),
          pltpu.SemaphoreType.DMA,
      ],
  )
  def deg_k(dst_hbm, ones_hbm, zeros_hbm, out_hbm, idx_v, ones_v, acc_sh, sem):
    c = lax.axis_index("c")
    s = lax.axis_index("s")
    w = c * NS + s
    pltpu.sync_copy(zeros_hbm.at[pl.ds(s * RPT, RPT)],
                    acc_sh.at[pl.ds(s * RPT, RPT)])
    pltpu.sync_copy(dst_hbm.at[w], idx_v)
    pltpu.sync_copy(ones_hbm, ones_v)
    plsc.subcore_barrier()

    @pl.loop(0, NCHUNK)
    def _(j):
      pltpu.sync_copy(ones_v, acc_sh.at[idx_v.at[j]], add=True)

    plsc.subcore_barrier()
    pltpu.sync_copy(acc_sh.at[pl.ds(s * RPT, RPT)],
                    out_hbm.at[c, pl.ds(s * RPT, RPT)])

  return deg_k(dst, ones, zeros)


def _sc_layer(table, src, dst, zeros):
  @functools.partial(
      pl.kernel,
      mesh=_mesh,
      out_type=jax.ShapeDtypeStruct((NC, N, DH), jnp.float32),
      compiler_params=_sc_params,
      scratch_types=[
          pltpu.VMEM((NCHUNK, CH), jnp.int32),
          pltpu.VMEM((NCHUNK, CH), jnp.int32),
          pltpu.VMEM((CH, DH), jnp.float32),
          pltpu.VMEM_SHARED((N, DH), jnp.float32),
          pltpu.SemaphoreType.DMA,
      ],
  )
  def layer_k(tab_hbm, src_hbm, dst_hbm, zeros_hbm, out_hbm,
              isrc_v, idst_v, rows_v, acc_sh, sem):
    c = lax.axis_index("c")
    s = lax.axis_index("s")
    w = c * NS + s
    pltpu.sync_copy(zeros_hbm.at[pl.ds(s * RPT, RPT)],
                    acc_sh.at[pl.ds(s * RPT, RPT)])
    pltpu.sync_copy(src_hbm.at[w], isrc_v)
    pltpu.sync_copy(dst_hbm.at[w], idst_v)
    plsc.subcore_barrier()

    @pl.loop(0, NCHUNK)
    def _(j):
      pltpu.async_copy(tab_hbm.at[isrc_v.at[j]], rows_v, sem).wait()
      pltpu.sync_copy(rows_v, acc_sh.at[idst_v.at[j]], add=True)

    plsc.subcore_barrier()
    pltpu.sync_copy(acc_sh.at[pl.ds(s * RPT, RPT)],
                    out_hbm.at[c, pl.ds(s * RPT, RPT)])

  return layer_k(table, src, dst, zeros)


def _tc_mm1(x, W1):
  def body(x_ref, w_ref, o_ref):
    o_ref[...] = jnp.dot(x_ref[...], w_ref[...],
                         preferred_element_type=jnp.float32)

  return pl.pallas_call(
      body, out_shape=jax.ShapeDtypeStruct((N, DH), jnp.float32))(x, W1)


def _tc_prep(p0, p1, h1):
  def body(p0_ref, p1_ref, h_ref, dinv_ref, g_ref):
    dinv = lax.rsqrt(p0_ref[...] + p1_ref[...] + 1.0)
    dinv_ref[...] = dinv
    g_ref[...] = h_ref[...] * dinv

  return pl.pallas_call(
      body,
      out_shape=[jax.ShapeDtypeStruct((N, DH), jnp.float32),
                 jax.ShapeDtypeStruct((N, DH), jnp.float32)])(p0, p1, h1)


def _tc_mid(p0, p1, g, dinv, b, W):
  def body(p0_ref, p1_ref, g_ref, dinv_ref, b_ref, w_ref, o_ref):
    z = dinv_ref[...] * (p0_ref[...] + p1_ref[...] + g_ref[...]) + b_ref[...]
    h = jnp.maximum(z, 0.0)
    o_ref[...] = jnp.dot(h, w_ref[...],
                         preferred_element_type=jnp.float32) * dinv_ref[...]

  return pl.pallas_call(
      body, out_shape=jax.ShapeDtypeStruct((N, DH), jnp.float32))(
          p0, p1, g, dinv, b, W)


def _tc_mid_noW(p0, p1, g, dinv, b):
  def body(p0_ref, p1_ref, g_ref, dinv_ref, b_ref, o_ref):
    z = dinv_ref[...] * (p0_ref[...] + p1_ref[...] + g_ref[...]) + b_ref[...]
    o_ref[...] = jnp.maximum(z, 0.0) * dinv_ref[...]

  return pl.pallas_call(
      body, out_shape=jax.ShapeDtypeStruct((N, DH), jnp.float32))(
          p0, p1, g, dinv, b)


def _tc_final(p0, p1, g, dinv, b3, W3):
  def body(p0_ref, p1_ref, g_ref, dinv_ref, b_ref, w_ref, o_ref):
    agg = dinv_ref[...] * (p0_ref[...] + p1_ref[...] + g_ref[...])
    z = jnp.dot(agg, w_ref[...], preferred_element_type=jnp.float32) + b_ref[...]
    m = jnp.max(z, axis=1, keepdims=True)
    lse = m + jnp.log(jnp.sum(jnp.exp(z - m), axis=1, keepdims=True))
    o_ref[...] = z - lse

  return pl.pallas_call(
      body, out_shape=jax.ShapeDtypeStruct((N, DOUT), jnp.float32))(
          p0, p1, g, dinv, b3, W3)


def kernel(x, edge_index, W1, b1, W2, b2, W3, b3):
  src = edge_index[0].astype(jnp.int32).reshape(NW, NCHUNK, CH)
  dst = edge_index[1].astype(jnp.int32).reshape(NW, NCHUNK, CH)
  ones = jnp.ones((CH, DH), jnp.float32)
  zeros = jnp.zeros((N, DH), jnp.float32)

  degp = _sc_degree(dst, ones, zeros)
  h1 = _tc_mm1(x, W1)
  dinv, g1 = _tc_prep(degp[0], degp[1], h1)

  p = _sc_layer(g1, src, dst, zeros)
  g2 = _tc_mid(p[0], p[1], g1, dinv, b1.reshape(1, DH), W2)

  p = _sc_layer(g2, src, dst, zeros)
  g3 = _tc_mid_noW(p[0], p[1], g2, dinv, b2.reshape(1, DH))

  p = _sc_layer(g3, src, dst, zeros)
  return _tc_final(p[0], p[1], g3, dinv, b3.reshape(1, DOUT), W3)


# speedup vs baseline: 47.0892x; 1.9005x over previous
import functools

import jax
import jax.numpy as jnp
from jax import lax
from jax.experimental import pallas as pl
from jax.experimental.pallas import tpu as pltpu
from jax.experimental.pallas import tpu_sc as plsc

N = 10000
E = 320000
DIN = 128
DH = 16
DOUT = 2
NC = 2
NS = 16
NW = NC * NS
EPW = E // NW
CH = 80
NCHUNK = EPW // CH
RPT = N // NS

_mesh = plsc.VectorSubcoreMesh(core_axis_name="c", subcore_axis_name="s")
_sc_params = pltpu.CompilerParams(use_tc_tiling_on_sc=False)


def _sc_degree(dst, ones, zeros):
  @functools.partial(
      pl.kernel,
      mesh=_mesh,
      out_type=jax.ShapeDtypeStruct((NC, N, DH), jnp.float32),
      compiler_params=_sc_params,
      scratch_types=[
          pltpu.VMEM((NCHUNK, CH), jnp.int32),
          pltpu.VMEM((CH, DH), jnp.float32),
          pltpu.VMEM_SHARED((N, DH), jnp.float32),
          pltpu.SemaphoreType.DMA,
      ],
  )
  def deg_k(dst_hbm, ones_hbm, zeros_hbm, out_hbm, idx_v, ones_v, acc_sh, sem):
    c = lax.axis_index("c")
    s = lax.axis_index("s")
    w = c * NS + s
    pltpu.sync_copy(zeros_hbm.at[pl.ds(s * RPT, RPT)],
                    acc_sh.at[pl.ds(s * RPT, RPT)])
    pltpu.sync_copy(dst_hbm.at[w], idx_v)
    pltpu.sync_copy(ones_hbm, ones_v)
    plsc.subcore_barrier()

    @pl.loop(0, NCHUNK)
    def _(j):
      pltpu.sync_copy(ones_v, acc_sh.at[idx_v.at[j]], add=True)

    plsc.subcore_barrier()
    pltpu.sync_copy(acc_sh.at[pl.ds(s * RPT, RPT)],
                    out_hbm.at[c, pl.ds(s * RPT, RPT)])

  return deg_k(dst, ones, zeros)


KG = 25
NG = NCHUNK // KG


def _sc_layer(table, src, dst, zeros):
  @functools.partial(
      pl.kernel,
      mesh=_mesh,
      out_type=jax.ShapeDtypeStruct((NC, N, DH), jnp.float32),
      compiler_params=_sc_params,
      scratch_types=[
          pltpu.VMEM((NCHUNK, CH), jnp.int32),
          pltpu.VMEM((NCHUNK, CH), jnp.int32),
          pltpu.VMEM((KG * CH, DH), jnp.float32),
          pltpu.VMEM((KG * CH, DH), jnp.float32),
          pltpu.VMEM_SHARED((N, DH), jnp.float32),
          pltpu.SemaphoreType.DMA,
          pltpu.SemaphoreType.DMA,
      ],
  )
  def layer_k(tab_hbm, src_hbm, dst_hbm, zeros_hbm, out_hbm,
              isrc_v, idst_v, buf_a, buf_b, acc_sh, sem_a, sem_b):
    c = lax.axis_index("c")
    s = lax.axis_index("s")
    w = c * NS + s
    pltpu.sync_copy(zeros_hbm.at[pl.ds(s * RPT, RPT)],
                    acc_sh.at[pl.ds(s * RPT, RPT)])
    pltpu.sync_copy(src_hbm.at[w], isrc_v)
    pltpu.sync_copy(dst_hbm.at[w], idst_v)
    plsc.subcore_barrier()

    def fire(g, buf, sem):
      @pl.loop(0, KG)
      def _(k):
        pltpu.async_copy(tab_hbm.at[isrc_v.at[g * KG + k]],
                         buf.at[pl.ds(k * CH, CH)], sem)

    def drain(buf, sem):
      pltpu.make_async_copy(tab_hbm.at[pl.ds(0, KG * CH)], buf, sem).wait()

    def scatter(g, buf):
      @pl.loop(0, KG)
      def _(k):
        pltpu.sync_copy(buf.at[pl.ds(k * CH, CH)],
                        acc_sh.at[idst_v.at[g * KG + k]], add=True)

    fire(0, buf_a, sem_a)
    for g in range(NG):
      cur, csem = (buf_a, sem_a) if g % 2 == 0 else (buf_b, sem_b)
      nxt, nsem = (buf_b, sem_b) if g % 2 == 0 else (buf_a, sem_a)
      if g + 1 < NG:
        fire(g + 1, nxt, nsem)
      drain(cur, csem)
      scatter(g, cur)

    plsc.subcore_barrier()
    pltpu.sync_copy(acc_sh.at[pl.ds(s * RPT, RPT)],
                    out_hbm.at[c, pl.ds(s * RPT, RPT)])

  return layer_k(table, src, dst, zeros)


def _tc_mm1(x, W1):
  def body(x_ref, w_ref, o_ref):
    o_ref[...] = jnp.dot(x_ref[...], w_ref[...],
                         preferred_element_type=jnp.float32)

  return pl.pallas_call(
      body, out_shape=jax.ShapeDtypeStruct((N, DH), jnp.float32))(x, W1)


def _tc_prep(p0, p1, h1):
  def body(p0_ref, p1_ref, h_ref, dinv_ref, g_ref):
    dinv = lax.rsqrt(p0_ref[...] + p1_ref[...] + 1.0)
    dinv_ref[...] = dinv
    g_ref[...] = h_ref[...] * dinv

  return pl.pallas_call(
      body,
      out_shape=[jax.ShapeDtypeStruct((N, DH), jnp.float32),
                 jax.ShapeDtypeStruct((N, DH), jnp.float32)])(p0, p1, h1)


def _tc_mid(p0, p1, g, dinv, b, W):
  def body(p0_ref, p1_ref, g_ref, dinv_ref, b_ref, w_ref, o_ref):
    z = dinv_ref[...] * (p0_ref[...] + p1_ref[...] + g_ref[...]) + b_ref[...]
    h = jnp.maximum(z, 0.0)
    o_ref[...] = jnp.dot(h, w_ref[...],
                         preferred_element_type=jnp.float32) * dinv_ref[...]

  return pl.pallas_call(
      body, out_shape=jax.ShapeDtypeStruct((N, DH), jnp.float32))(
          p0, p1, g, dinv, b, W)


def _tc_mid_noW(p0, p1, g, dinv, b):
  def body(p0_ref, p1_ref, g_ref, dinv_ref, b_ref, o_ref):
    z = dinv_ref[...] * (p0_ref[...] + p1_ref[...] + g_ref[...]) + b_ref[...]
    o_ref[...] = jnp.maximum(z, 0.0) * dinv_ref[...]

  return pl.pallas_call(
      body, out_shape=jax.ShapeDtypeStruct((N, DH), jnp.float32))(
          p0, p1, g, dinv, b)


def _tc_final(p0, p1, g, dinv, b3, W3):
  def body(p0_ref, p1_ref, g_ref, dinv_ref, b_ref, w_ref, o_ref):
    agg = dinv_ref[...] * (p0_ref[...] + p1_ref[...] + g_ref[...])
    z = jnp.dot(agg, w_ref[...], preferred_element_type=jnp.float32) + b_ref[...]
    m = jnp.max(z, axis=1, keepdims=True)
    lse = m + jnp.log(jnp.sum(jnp.exp(z - m), axis=1, keepdims=True))
    o_ref[...] = z - lse

  return pl.pallas_call(
      body, out_shape=jax.ShapeDtypeStruct((N, DOUT), jnp.float32))(
          p0, p1, g, dinv, b3, W3)


def kernel(x, edge_index, W1, b1, W2, b2, W3, b3):
  src = edge_index[0].astype(jnp.int32).reshape(NW, NCHUNK, CH)
  dst = edge_index[1].astype(jnp.int32).reshape(NW, NCHUNK, CH)
  ones = jnp.ones((CH, DH), jnp.float32)
  zeros = jnp.zeros((N, DH), jnp.float32)

  degp = _sc_degree(dst, ones, zeros)
  h1 = _tc_mm1(x, W1)
  dinv, g1 = _tc_prep(degp[0], degp[1], h1)

  p = _sc_layer(g1, src, dst, zeros)
  g2 = _tc_mid(p[0], p[1], g1, dinv, b1.reshape(1, DH), W2)

  p = _sc_layer(g2, src, dst, zeros)
  g3 = _tc_mid_noW(p[0], p[1], g2, dinv, b2.reshape(1, DH))

  p = _sc_layer(g3, src, dst, zeros)
  return _tc_final(p[0], p[1], g3, dinv, b3.reshape(1, DOUT), W3)


# speedup vs baseline: 85.5764x; 1.8173x over previous
import functools

import jax
import jax.numpy as jnp
from jax import lax
from jax.experimental import pallas as pl
from jax.experimental.pallas import tpu as pltpu
from jax.experimental.pallas import tpu_sc as plsc

N = 10000
NP = 10240
E = 320000
DIN = 128
DH = 16
DOUT = 2
NC = 2
NS = 16
NW = NC * NS
CH = 128
NCH = E // CH
CPW = NCH // NW
REM = NCH - CPW * NW
KG = 13
NG = CPW // KG
RPT = NP // NS
PR = NP * DH // 128
PRV = N * DH // 128

_mesh = plsc.VectorSubcoreMesh(core_axis_name="c", subcore_axis_name="s")
_sc_params = pltpu.CompilerParams(use_tc_tiling_on_sc=False)

_f32 = jnp.float32


def _sc_degree(dst2, ones, zeros):
  @functools.partial(
      pl.kernel,
      mesh=_mesh,
      out_type=jax.ShapeDtypeStruct((NC, NP, DH), _f32),
      compiler_params=_sc_params,
      scratch_types=[
          pltpu.VMEM((CPW + 1, CH), jnp.int32),
          pltpu.VMEM((CH, DH), _f32),
          pltpu.VMEM_SHARED((NP, DH), _f32),
          pltpu.SemaphoreType.DMA,
          pltpu.SemaphoreType.DMA,
      ],
  )
  def deg_k(dst_hbm, ones_hbm, zeros_hbm, out_hbm, idx_v, ones_v, acc_sh,
            sem, sem2):
    c = lax.axis_index("c")
    s = lax.axis_index("s")
    w = c * NS + s

    @pl.loop(0, CPW)
    def _(j):
      pltpu.async_copy(dst_hbm.at[w + NW * j], idx_v.at[j], sem)

    pltpu.sync_copy(zeros_hbm.at[pl.ds(s * RPT, RPT)],
                    acc_sh.at[pl.ds(s * RPT, RPT)])
    pltpu.sync_copy(ones_hbm, ones_v)
    pltpu.make_async_copy(dst_hbm.at[pl.ds(0, CPW)],
                          idx_v.at[pl.ds(0, CPW)], sem).wait()
    extra = w < REM

    @pl.when(extra)
    def _():
      pltpu.async_copy(dst_hbm.at[NW * CPW + w], idx_v.at[CPW], sem2).wait()

    plsc.subcore_barrier()

    @pl.loop(0, CPW)
    def _(j):
      pltpu.sync_copy(ones_v, acc_sh.at[idx_v.at[j]], add=True)

    @pl.when(extra)
    def _():
      pltpu.sync_copy(ones_v, acc_sh.at[idx_v.at[CPW]], add=True)

    plsc.subcore_barrier()
    pltpu.sync_copy(acc_sh.at[pl.ds(s * RPT, RPT)],
                    out_hbm.at[c, pl.ds(s * RPT, RPT)])

  return deg_k(dst2, ones, zeros)


def _sc_layer(table, src2, dst2, zeros):
  @functools.partial(
      pl.kernel,
      mesh=_mesh,
      out_type=jax.ShapeDtypeStruct((NC, NP, DH), _f32),
      compiler_params=_sc_params,
      scratch_types=[
          pltpu.VMEM((CPW + 1, CH), jnp.int32),
          pltpu.VMEM((CPW + 1, CH), jnp.int32),
          pltpu.VMEM((KG * CH, DH), _f32),
          pltpu.VMEM((KG * CH, DH), _f32),
          pltpu.VMEM_SHARED((NP, DH), _f32),
          pltpu.SemaphoreType.DMA,
          pltpu.SemaphoreType.DMA,
          pltpu.SemaphoreType.DMA,
          pltpu.SemaphoreType.DMA,
      ],
  )
  def layer_k(tab_hbm, src_hbm, dst_hbm, zeros_hbm, out_hbm,
              isrc_v, idst_v, buf_a, buf_b, acc_sh, sidx, sa, sb, sx):
    c = lax.axis_index("c")
    s = lax.axis_index("s")
    w = c * NS + s
    tab = tab_hbm

    @pl.loop(0, CPW)
    def _(j):
      pltpu.async_copy(src_hbm.at[w + NW * j], isrc_v.at[j], sidx)
      pltpu.async_copy(dst_hbm.at[w + NW * j], idst_v.at[j], sidx)

    pltpu.sync_copy(zeros_hbm.at[pl.ds(s * RPT, RPT)],
                    acc_sh.at[pl.ds(s * RPT, RPT)])
    pltpu.make_async_copy(src_hbm.at[pl.ds(0, CPW)],
                          isrc_v.at[pl.ds(0, CPW)], sidx).wait()
    pltpu.make_async_copy(dst_hbm.at[pl.ds(0, CPW)],
                          idst_v.at[pl.ds(0, CPW)], sidx).wait()
    extra = w < REM

    @pl.when(extra)
    def _():
      pltpu.async_copy(src_hbm.at[NW * CPW + w], isrc_v.at[CPW], sx).wait()
      pltpu.async_copy(dst_hbm.at[NW * CPW + w], idst_v.at[CPW], sx).wait()

    plsc.subcore_barrier()

    def fire(g, buf, sem):
      @pl.loop(0, KG)
      def _(k):
        pltpu.async_copy(tab.at[isrc_v.at[g * KG + k]],
                         buf.at[pl.ds(k * CH, CH)], sem)

    def drain(buf, sem):
      pltpu.make_async_copy(tab.at[pl.ds(0, KG * CH)], buf, sem).wait()

    def scatter(g, buf):
      @pl.loop(0, KG)
      def _(k):
        pltpu.sync_copy(buf.at[pl.ds(k * CH, CH)],
                        acc_sh.at[idst_v.at[g * KG + k]], add=True)

    fire(0, buf_a, sa)
    for g in range(NG):
      cur, csem = (buf_a, sa) if g % 2 == 0 else (buf_b, sb)
      nxt, nsem = (buf_b, sb) if g % 2 == 0 else (buf_a, sa)
      if g + 1 < NG:
        fire(g + 1, nxt, nsem)
      drain(cur, csem)
      scatter(g, cur)

    @pl.when(extra)
    def _():
      pltpu.async_copy(tab.at[isrc_v.at[CPW]],
                       buf_a.at[pl.ds(0, CH)], sx).wait()
      pltpu.sync_copy(buf_a.at[pl.ds(0, CH)],
                      acc_sh.at[idst_v.at[CPW]], add=True)

    plsc.subcore_barrier()
    pltpu.sync_copy(acc_sh.at[pl.ds(s * RPT, RPT)],
                    out_hbm.at[c, pl.ds(s * RPT, RPT)])

  return layer_k(table, src2, dst2, zeros)


def _tc_mm1(xp, W1k):
  def body(x_ref, w_ref, o_ref):
    o_ref[0:PRV, :] = jnp.dot(x_ref[...], w_ref[...],
                              preferred_element_type=_f32)
    o_ref[PRV:PR, :] = jnp.zeros((PR - PRV, 128), _f32)

  return pl.pallas_call(
      body, out_shape=jax.ShapeDtypeStruct((PR, 128), _f32))(xp, W1k)


def _tc_prep(p, h1):
  def body(p_ref, h_ref, dinv_ref, g_ref):
    dinv = lax.rsqrt(p_ref[0] + p_ref[1] + 1.0)
    dinv_ref[...] = dinv
    g_ref[...] = h_ref[...] * dinv

  return pl.pallas_call(
      body,
      out_shape=[jax.ShapeDtypeStruct((PR, 128), _f32),
                 jax.ShapeDtypeStruct((PR, 128), _f32)])(p, h1)


def _tc_mid(p, g, dinv, bp, Wk):
  def body(p_ref, g_ref, dinv_ref, b_ref, w_ref, o_ref):
    z = dinv_ref[...] * (p_ref[0] + p_ref[1] + g_ref[...]) + b_ref[...]
    h = jnp.maximum(z, 0.0)
    o_ref[...] = jnp.dot(h, w_ref[...],
                         preferred_element_type=_f32) * dinv_ref[...]

  return pl.pallas_call(
      body, out_shape=jax.ShapeDtypeStruct((PR, 128), _f32))(
          p, g, dinv, bp, Wk)


def _tc_mid_noW(p, g, dinv, bp):
  def body(p_ref, g_ref, dinv_ref, b_ref, o_ref):
    z = dinv_ref[...] * (p_ref[0] + p_ref[1] + g_ref[...]) + b_ref[...]
    o_ref[...] = jnp.maximum(z, 0.0) * dinv_ref[...]

  return pl.pallas_call(
      body, out_shape=jax.ShapeDtypeStruct((PR, 128), _f32))(p, g, dinv, bp)


def _tc_final(p, g, dinv, b3p, W3k, P16):
  def body(p_ref, g_ref, dinv_ref, b_ref, w_ref, perm_ref, o_ref):
    agg = dinv_ref[...] * (p_ref[0] + p_ref[1] + g_ref[...])
    aggv = agg[0:PRV, :]
    z = jnp.dot(aggv, w_ref[...], preferred_element_type=_f32) + b_ref[...]
    zs = jnp.dot(z, perm_ref[...], preferred_element_type=_f32)
    m = jnp.maximum(z, zs)
    o_ref[...] = z - (m + jnp.log(jnp.exp(z - m) + jnp.exp(zs - m)))

  return pl.pallas_call(
      body, out_shape=jax.ShapeDtypeStruct((PRV, 8 * DOUT), _f32))(
          p, g, dinv, b3p, W3k, P16)


def kernel(x, edge_index, W1, b1, W2, b2, W3, b3):
  src2 = edge_index[0].astype(jnp.int32).reshape(NCH, CH)
  dst2 = edge_index[1].astype(jnp.int32).reshape(NCH, CH)
  xp = x.reshape(PRV, 8 * DIN)
  ones = jnp.ones((CH, DH), _f32)
  zeros = jnp.zeros((NP, DH), _f32)
  eye8 = jnp.eye(8, dtype=_f32)
  W1k = jnp.kron(eye8, W1)
  W2k = jnp.kron(eye8, W2)
  W3k = jnp.kron(eye8, W3)
  P16 = jnp.kron(eye8, jnp.array([[0.0, 1.0], [1.0, 0.0]], _f32))
  b1p = jnp.tile(b1, 8).reshape(1, 128)
  b2p = jnp.tile(b2, 8).reshape(1, 128)
  b3p = jnp.tile(b3, 8).reshape(1, 8 * DOUT)

  def packed(a):
    return a.reshape(NC, PR, 128)

  def flat(gp):
    return gp.reshape(NP, DH)

  degp = _sc_degree(dst2, ones, zeros)
  h1 = _tc_mm1(xp, W1k)
  dinv, g1 = _tc_prep(packed(degp), h1)

  p = _sc_layer(flat(g1), src2, dst2, zeros)
  g2 = _tc_mid(packed(p), g1, dinv, b1p, W2k)

  p = _sc_layer(flat(g2), src2, dst2, zeros)
  g3 = _tc_mid_noW(packed(p), g2, dinv, b2p)

  p = _sc_layer(flat(g3), src2, dst2, zeros)
  outp = _tc_final(packed(p), g3, dinv, b3p, W3k, P16)
  return outp.reshape(N, DOUT)


# speedup vs baseline: 94.4805x; 1.1040x over previous
import functools

import jax
import jax.numpy as jnp
from jax import lax
from jax.experimental import pallas as pl
from jax.experimental.pallas import tpu as pltpu
from jax.experimental.pallas import tpu_sc as plsc

N = 10000
NP = 10240
E = 320000
DIN = 128
DH = 16
DOUT = 2
NC = 2
NS = 16
NW = NC * NS
CH = 128
NCH = E // CH
CPW = NCH // NW
REM = NCH - CPW * NW
KG = 13
NG = CPW // KG
RPT = NP // NS
PR = NP * DH // 128
PRV = N * DH // 128

_mesh = plsc.VectorSubcoreMesh(core_axis_name="c", subcore_axis_name="s")
_sc_params = pltpu.CompilerParams(use_tc_tiling_on_sc=False)

_f32 = jnp.float32


def _sc_degree(ei3, ones, zeros):
  @functools.partial(
      pl.kernel,
      mesh=_mesh,
      out_type=jax.ShapeDtypeStruct((NC, NP, DH), _f32),
      compiler_params=_sc_params,
      scratch_types=[
          pltpu.VMEM((CPW + 1, 2, CH), jnp.int32),
          pltpu.VMEM((CH, DH), _f32),
          pltpu.VMEM_SHARED((NP, DH), _f32),
          pltpu.SemaphoreType.DMA,
          pltpu.SemaphoreType.DMA,
      ],
  )
  def deg_k(ei_hbm, ones_hbm, zeros_hbm, out_hbm, idx_v, ones_v, acc_sh,
            sem, sem2):
    c = lax.axis_index("c")
    s = lax.axis_index("s")
    w = c * NS + s

    @pl.loop(0, CPW)
    def _(j):
      pltpu.async_copy(ei_hbm.at[w + NW * j], idx_v.at[j], sem)

    pltpu.sync_copy(zeros_hbm.at[pl.ds(s * RPT, RPT)],
                    acc_sh.at[pl.ds(s * RPT, RPT)])
    pltpu.sync_copy(ones_hbm, ones_v)
    pltpu.make_async_copy(ei_hbm.at[pl.ds(0, CPW)],
                          idx_v.at[pl.ds(0, CPW)], sem).wait()
    extra = w < REM

    @pl.when(extra)
    def _():
      pltpu.async_copy(ei_hbm.at[NW * CPW + w], idx_v.at[CPW], sem2).wait()

    plsc.subcore_barrier()

    @pl.loop(0, CPW)
    def _(j):
      pltpu.sync_copy(ones_v, acc_sh.at[idx_v.at[j, 1]], add=True)

    @pl.when(extra)
    def _():
      pltpu.sync_copy(ones_v, acc_sh.at[idx_v.at[CPW, 1]], add=True)

    plsc.subcore_barrier()
    pltpu.sync_copy(acc_sh.at[pl.ds(s * RPT, RPT)],
                    out_hbm.at[c, pl.ds(s * RPT, RPT)])

  return deg_k(ei3, ones, zeros)


def _sc_layer(table, ei3, zeros):
  @functools.partial(
      pl.kernel,
      mesh=_mesh,
      out_type=jax.ShapeDtypeStruct((NC, NP, DH), _f32),
      compiler_params=_sc_params,
      scratch_types=[
          pltpu.VMEM((CPW + 1, 2, CH), jnp.int32),
          pltpu.VMEM((KG * CH, DH), _f32),
          pltpu.VMEM((KG * CH, DH), _f32),
          pltpu.VMEM_SHARED((NP, DH), _f32),
          pltpu.SemaphoreType.DMA,
          pltpu.SemaphoreType.DMA,
          pltpu.SemaphoreType.DMA,
          pltpu.SemaphoreType.DMA,
      ],
  )
  def layer_k(tab_hbm, ei_hbm, zeros_hbm, out_hbm,
              idx_v, buf_a, buf_b, acc_sh, sidx, sa, sb, sx):
    c = lax.axis_index("c")
    s = lax.axis_index("s")
    w = c * NS + s
    tab = tab_hbm

    @pl.loop(0, CPW)
    def _(j):
      pltpu.async_copy(ei_hbm.at[w + NW * j], idx_v.at[j], sidx)

    pltpu.sync_copy(zeros_hbm.at[pl.ds(s * RPT, RPT)],
                    acc_sh.at[pl.ds(s * RPT, RPT)])
    pltpu.make_async_copy(ei_hbm.at[pl.ds(0, CPW)],
                          idx_v.at[pl.ds(0, CPW)], sidx).wait()
    extra = w < REM

    @pl.when(extra)
    def _():
      pltpu.async_copy(ei_hbm.at[NW * CPW + w], idx_v.at[CPW], sx).wait()

    plsc.subcore_barrier()

    def fire(g, buf, sem):
      @pl.loop(0, KG)
      def _(k):
        pltpu.async_copy(tab.at[idx_v.at[g * KG + k, 0]],
                         buf.at[pl.ds(k * CH, CH)], sem)

    def drain(buf, sem):
      pltpu.make_async_copy(tab.at[pl.ds(0, KG * CH)], buf, sem).wait()

    def scatter(g, buf):
      @pl.loop(0, KG)
      def _(k):
        pltpu.sync_copy(buf.at[pl.ds(k * CH, CH)],
                        acc_sh.at[idx_v.at[g * KG + k, 1]], add=True)

    fire(0, buf_a, sa)
    for g in range(NG):
      cur, csem = (buf_a, sa) if g % 2 == 0 else (buf_b, sb)
      nxt, nsem = (buf_b, sb) if g % 2 == 0 else (buf_a, sa)
      if g + 1 < NG:
        fire(g + 1, nxt, nsem)
      drain(cur, csem)
      scatter(g, cur)

    @pl.when(extra)
    def _():
      pltpu.async_copy(tab.at[idx_v.at[CPW, 0]],
                       buf_a.at[pl.ds(0, CH)], sx).wait()
      pltpu.sync_copy(buf_a.at[pl.ds(0, CH)],
                      acc_sh.at[idx_v.at[CPW, 1]], add=True)

    plsc.subcore_barrier()
    pltpu.sync_copy(acc_sh.at[pl.ds(s * RPT, RPT)],
                    out_hbm.at[c, pl.ds(s * RPT, RPT)])

  return layer_k(table, ei3, zeros)


def _tc_mm1(xp, W1k):
  def body(x_ref, w_ref, o_ref):
    o_ref[0:PRV, :] = jnp.dot(x_ref[...], w_ref[...],
                              preferred_element_type=_f32)
    o_ref[PRV:PR, :] = jnp.zeros((PR - PRV, 128), _f32)

  return pl.pallas_call(
      body, out_shape=jax.ShapeDtypeStruct((PR, 128), _f32))(xp, W1k)


def _tc_prep(p, h1):
  def body(p_ref, h_ref, dinv_ref, g_ref):
    dinv = lax.rsqrt(p_ref[0] + p_ref[1] + 1.0)
    dinv_ref[...] = dinv
    g_ref[...] = h_ref[...] * dinv

  return pl.pallas_call(
      body,
      out_shape=[jax.ShapeDtypeStruct((PR, 128), _f32),
                 jax.ShapeDtypeStruct((PR, 128), _f32)])(p, h1)


def _tc_mid(p, g, dinv, bp, Wk):
  def body(p_ref, g_ref, dinv_ref, b_ref, w_ref, o_ref):
    z = dinv_ref[...] * (p_ref[0] + p_ref[1] + g_ref[...]) + b_ref[...]
    h = jnp.maximum(z, 0.0)
    o_ref[...] = jnp.dot(h, w_ref[...],
                         preferred_element_type=_f32) * dinv_ref[...]

  return pl.pallas_call(
      body, out_shape=jax.ShapeDtypeStruct((PR, 128), _f32))(
          p, g, dinv, bp, Wk)


def _tc_mid_noW(p, g, dinv, bp):
  def body(p_ref, g_ref, dinv_ref, b_ref, o_ref):
    z = dinv_ref[...] * (p_ref[0] + p_ref[1] + g_ref[...]) + b_ref[...]
    o_ref[...] = jnp.maximum(z, 0.0) * dinv_ref[...]

  return pl.pallas_call(
      body, out_shape=jax.ShapeDtypeStruct((PR, 128), _f32))(p, g, dinv, bp)


def _tc_final(p, g, dinv, b3p, W3k, P16):
  def body(p_ref, g_ref, dinv_ref, b_ref, w_ref, perm_ref, o_ref):
    agg = dinv_ref[...] * (p_ref[0] + p_ref[1] + g_ref[...])
    aggv = agg[0:PRV, :]
    z = jnp.dot(aggv, w_ref[...], preferred_element_type=_f32) + b_ref[...]
    zs = jnp.dot(z, perm_ref[...], preferred_element_type=_f32)
    m = jnp.maximum(z, zs)
    o_ref[...] = z - (m + jnp.log(jnp.exp(z - m) + jnp.exp(zs - m)))

  return pl.pallas_call(
      body, out_shape=jax.ShapeDtypeStruct((PRV, 8 * DOUT), _f32))(
          p, g, dinv, b3p, W3k, P16)


def kernel(x, edge_index, W1, b1, W2, b2, W3, b3):
  ei3 = edge_index.astype(jnp.int32).reshape(2, NCH, CH).transpose(1, 0, 2)
  xp = x.reshape(PRV, 8 * DIN)
  ones = jnp.ones((CH, DH), _f32)
  zeros = jnp.zeros((NP, DH), _f32)
  eye8 = jnp.eye(8, dtype=_f32)
  W1k = jnp.kron(eye8, W1)
  W2k = jnp.kron(eye8, W2)
  W3k = jnp.kron(eye8, W3)
  P16 = jnp.kron(eye8, jnp.array([[0.0, 1.0], [1.0, 0.0]], _f32))
  b1p = jnp.tile(b1, 8).reshape(1, 128)
  b2p = jnp.tile(b2, 8).reshape(1, 128)
  b3p = jnp.tile(b3, 8).reshape(1, 8 * DOUT)

  def packed(a):
    return a.reshape(NC, PR, 128)

  def flat(gp):
    return gp.reshape(NP, DH)

  degp = _sc_degree(ei3, ones, zeros)
  h1 = _tc_mm1(xp, W1k)
  dinv, g1 = _tc_prep(packed(degp), h1)

  p = _sc_layer(flat(g1), ei3, zeros)
  g2 = _tc_mid(packed(p), g1, dinv, b1p, W2k)

  p = _sc_layer(flat(g2), ei3, zeros)
  g3 = _tc_mid_noW(packed(p), g2, dinv, b2p)

  p = _sc_layer(flat(g3), ei3, zeros)
  outp = _tc_final(packed(p), g3, dinv, b3p, W3k, P16)
  return outp.reshape(N, DOUT)


# speedup vs baseline: 99.8251x; 1.0566x over previous
import functools

import jax
import jax.numpy as jnp
from jax import lax
from jax.experimental import pallas as pl
from jax.experimental.pallas import tpu as pltpu
from jax.experimental.pallas import tpu_sc as plsc

N = 10000
NP = 10240
E = 320000
DIN = 128
DH = 16
DOUT = 2
NC = 2
NS = 16
NW = NC * NS
CH = 128
NCH = E // CH
CPW = NCH // NW
REM = NCH - CPW * NW
KG = 13
NG = CPW // KG
RPT = NP // NS
PR = NP * DH // 128
PRV = N * DH // 128

_mesh = plsc.VectorSubcoreMesh(core_axis_name="c", subcore_axis_name="s")
_sc_params = pltpu.CompilerParams(use_tc_tiling_on_sc=False)

_f32 = jnp.float32


def _sc_degree(ei3, ones, zeros):
  @functools.partial(
      pl.kernel,
      mesh=_mesh,
      out_type=jax.ShapeDtypeStruct((NC, NP, DH), _f32),
      compiler_params=_sc_params,
      scratch_types=[
          pltpu.VMEM((CPW + 1, 2, CH), jnp.int32),
          pltpu.VMEM((CH, DH), _f32),
          pltpu.VMEM_SHARED((NP, DH), _f32),
          pltpu.SemaphoreType.DMA,
          pltpu.SemaphoreType.DMA,
      ],
  )
  def deg_k(ei_hbm, ones_hbm, zeros_hbm, out_hbm, idx_v, ones_v, acc_sh,
            sem, sem2):
    c = lax.axis_index("c")
    s = lax.axis_index("s")
    w = c * NS + s

    @pl.loop(0, CPW)
    def _(j):
      pltpu.async_copy(ei_hbm.at[w + NW * j], idx_v.at[j], sem)

    pltpu.sync_copy(zeros_hbm.at[pl.ds(s * RPT, RPT)],
                    acc_sh.at[pl.ds(s * RPT, RPT)])
    pltpu.sync_copy(ones_hbm, ones_v)
    pltpu.make_async_copy(ei_hbm.at[pl.ds(0, CPW)],
                          idx_v.at[pl.ds(0, CPW)], sem).wait()
    extra = w < REM

    @pl.when(extra)
    def _():
      pltpu.async_copy(ei_hbm.at[NW * CPW + w], idx_v.at[CPW], sem2).wait()

    plsc.subcore_barrier()

    hs = [pltpu.async_copy(ones_v, acc_sh.at[idx_v.at[j, 1]], sem, add=True)
          for j in range(CPW)]

    @pl.when(extra)
    def _():
      pltpu.sync_copy(ones_v, acc_sh.at[idx_v.at[CPW, 1]], add=True)

    for h in hs:
      h.wait()

    plsc.subcore_barrier()
    pltpu.sync_copy(acc_sh.at[pl.ds(s * RPT, RPT)],
                    out_hbm.at[c, pl.ds(s * RPT, RPT)])

  return deg_k(ei3, ones, zeros)


def _sc_layer(table, ei3, zeros):
  @functools.partial(
      pl.kernel,
      mesh=_mesh,
      out_type=jax.ShapeDtypeStruct((NC, NP, DH), _f32),
      compiler_params=_sc_params,
      scratch_types=[
          pltpu.VMEM((CPW + 1, 2, CH), jnp.int32),
          pltpu.VMEM((KG * CH, DH), _f32),
          pltpu.VMEM((KG * CH, DH), _f32),
          pltpu.VMEM_SHARED((NP, DH), _f32),
          pltpu.SemaphoreType.DMA,
          pltpu.SemaphoreType.DMA,
          pltpu.SemaphoreType.DMA,
          pltpu.SemaphoreType.DMA,
      ],
  )
  def layer_k(tab_hbm, ei_hbm, zeros_hbm, out_hbm,
              idx_v, buf_a, buf_b, acc_sh, sidx, sa, sb, sx):
    c = lax.axis_index("c")
    s = lax.axis_index("s")
    w = c * NS + s
    tab = tab_hbm

    @pl.loop(0, CPW)
    def _(j):
      pltpu.async_copy(ei_hbm.at[w + NW * j], idx_v.at[j], sidx)

    pltpu.sync_copy(zeros_hbm.at[pl.ds(s * RPT, RPT)],
                    acc_sh.at[pl.ds(s * RPT, RPT)])
    pltpu.make_async_copy(ei_hbm.at[pl.ds(0, CPW)],
                          idx_v.at[pl.ds(0, CPW)], sidx).wait()
    extra = w < REM

    @pl.when(extra)
    def _():
      pltpu.async_copy(ei_hbm.at[NW * CPW + w], idx_v.at[CPW], sx).wait()

    plsc.subcore_barrier()

    def fire(g, buf, sem):
      @pl.loop(0, KG)
      def _(k):
        pltpu.async_copy(tab.at[idx_v.at[g * KG + k, 0]],
                         buf.at[pl.ds(k * CH, CH)], sem)

    def drain(buf, sem):
      pltpu.make_async_copy(tab.at[pl.ds(0, KG * CH)], buf, sem).wait()

    def scatter(g, buf, sem):
      return [pltpu.async_copy(buf.at[pl.ds(k * CH, CH)],
                               acc_sh.at[idx_v.at[g * KG + k, 1]],
                               sem, add=True)
              for k in range(KG)]

    fire(0, buf_a, sa)
    pend = []
    for g in range(NG):
      cur, csem = (buf_a, sa) if g % 2 == 0 else (buf_b, sb)
      nxt, nsem = (buf_b, sb) if g % 2 == 0 else (buf_a, sa)
      for h in pend:
        h.wait()
      if g + 1 < NG:
        fire(g + 1, nxt, nsem)
      drain(cur, csem)
      pend = scatter(g, cur, sx)
    for h in pend:
      h.wait()

    @pl.when(extra)
    def _():
      pltpu.async_copy(tab.at[idx_v.at[CPW, 0]],
                       buf_a.at[pl.ds(0, CH)], sx).wait()
      pltpu.sync_copy(buf_a.at[pl.ds(0, CH)],
                      acc_sh.at[idx_v.at[CPW, 1]], add=True)

    plsc.subcore_barrier()
    pltpu.sync_copy(acc_sh.at[pl.ds(s * RPT, RPT)],
                    out_hbm.at[c, pl.ds(s * RPT, RPT)])

  return layer_k(table, ei3, zeros)


def _tc_mm1(xp, W1k):
  def body(x_ref, w_ref, o_ref):
    o_ref[0:PRV, :] = jnp.dot(x_ref[...], w_ref[...],
                              preferred_element_type=_f32)
    o_ref[PRV:PR, :] = jnp.zeros((PR - PRV, 128), _f32)

  return pl.pallas_call(
      body, out_shape=jax.ShapeDtypeStruct((PR, 128), _f32))(xp, W1k)


def _tc_prep(p, h1):
  def body(p_ref, h_ref, dinv_ref, g_ref):
    dinv = lax.rsqrt(p_ref[0] + p_ref[1] + 1.0)
    dinv_ref[...] = dinv
    g_ref[...] = h_ref[...] * dinv

  return pl.pallas_call(
      body,
      out_shape=[jax.ShapeDtypeStruct((PR, 128), _f32),
                 jax.ShapeDtypeStruct((PR, 128), _f32)])(p, h1)


def _tc_mid(p, g, dinv, bp, Wk):
  def body(p_ref, g_ref, dinv_ref, b_ref, w_ref, o_ref):
    z = dinv_ref[...] * (p_ref[0] + p_ref[1] + g_ref[...]) + b_ref[...]
    h = jnp.maximum(z, 0.0)
    o_ref[...] = jnp.dot(h, w_ref[...],
                         preferred_element_type=_f32) * dinv_ref[...]

  return pl.pallas_call(
      body, out_shape=jax.ShapeDtypeStruct((PR, 128), _f32))(
          p, g, dinv, bp, Wk)


def _tc_mid_noW(p, g, dinv, bp):
  def body(p_ref, g_ref, dinv_ref, b_ref, o_ref):
    z = dinv_ref[...] * (p_ref[0] + p_ref[1] + g_ref[...]) + b_ref[...]
    o_ref[...] = jnp.maximum(z, 0.0) * dinv_ref[...]

  return pl.pallas_call(
      body, out_shape=jax.ShapeDtypeStruct((PR, 128), _f32))(p, g, dinv, bp)


def _tc_final(p, g, dinv, b3p, W3k, P16):
  def body(p_ref, g_ref, dinv_ref, b_ref, w_ref, perm_ref, o_ref):
    agg = dinv_ref[...] * (p_ref[0] + p_ref[1] + g_ref[...])
    aggv = agg[0:PRV, :]
    z = jnp.dot(aggv, w_ref[...], preferred_element_type=_f32) + b_ref[...]
    zs = jnp.dot(z, perm_ref[...], preferred_element_type=_f32)
    m = jnp.maximum(z, zs)
    o_ref[...] = z - (m + jnp.log(jnp.exp(z - m) + jnp.exp(zs - m)))

  return pl.pallas_call(
      body, out_shape=jax.ShapeDtypeStruct((PRV, 8 * DOUT), _f32))(
          p, g, dinv, b3p, W3k, P16)


def kernel(x, edge_index, W1, b1, W2, b2, W3, b3):
  ei3 = edge_index.astype(jnp.int32).reshape(2, NCH, CH).transpose(1, 0, 2)
  xp = x.reshape(PRV, 8 * DIN)
  ones = jnp.ones((CH, DH), _f32)
  zeros = jnp.zeros((NP, DH), _f32)
  eye8 = jnp.eye(8, dtype=_f32)
  W1k = jnp.kron(eye8, W1)
  W2k = jnp.kron(eye8, W2)
  W3k = jnp.kron(eye8, W3)
  P16 = jnp.kron(eye8, jnp.array([[0.0, 1.0], [1.0, 0.0]], _f32))
  b1p = jnp.tile(b1, 8).reshape(1, 128)
  b2p = jnp.tile(b2, 8).reshape(1, 128)
  b3p = jnp.tile(b3, 8).reshape(1, 8 * DOUT)

  def packed(a):
    return a.reshape(NC, PR, 128)

  def flat(gp):
    return gp.reshape(NP, DH)

  degp = _sc_degree(ei3, ones, zeros)
  h1 = _tc_mm1(xp, W1k)
  dinv, g1 = _tc_prep(packed(degp), h1)

  p = _sc_layer(flat(g1), ei3, zeros)
  g2 = _tc_mid(packed(p), g1, dinv, b1p, W2k)

  p = _sc_layer(flat(g2), ei3, zeros)
  g3 = _tc_mid_noW(packed(p), g2, dinv, b2p)

  p = _sc_layer(flat(g3), ei3, zeros)
  outp = _tc_final(packed(p), g3, dinv, b3p, W3k, P16)
  return outp.reshape(N, DOUT)
